# baseline (speedup 1.0000x reference)
import functools

import numpy as np

import jax
from jax import lax
import jax.numpy as jnp
from jax.experimental import pallas as pl
from jax.experimental.pallas import tpu as pltpu
from jax.experimental.pallas import tpu_sc as plsc

BATCH = 64
MAX_SEQ = 2048
D_MODEL = 768
VOCAB = 44
NTOK = BATCH * MAX_SEQ
NW = 32
P_PER_W = MAX_SEQ // NW


def _positional_encoding(d_model, max_len):
    position = jnp.arange(0, max_len, dtype=jnp.float32)[:, None]
    div_term = jnp.exp(
        jnp.arange(0, d_model, 2, dtype=jnp.float32) * (-np.log(10000.0) / d_model)
    )
    pe = jnp.zeros((max_len, d_model), dtype=jnp.float32)
    pe = pe.at[:, 0::2].set(jnp.sin(position * div_term))
    pe = pe.at[:, 1::2].set(jnp.cos(position * div_term))
    return pe


@functools.partial(jax.jit, static_argnums=())
def kernel(tokens, emb_table):
    pe = _positional_encoding(D_MODEL, MAX_SEQ)
    idx = tokens.reshape(NTOK)

    mesh = plsc.VectorSubcoreMesh(core_axis_name="c", subcore_axis_name="s")

    @functools.partial(
        pl.kernel,
        mesh=mesh,
        out_type=jax.ShapeDtypeStruct((NTOK, D_MODEL), jnp.float32),
        scratch_types=[
            pltpu.VMEM((P_PER_W,), jnp.int32),
            pltpu.VMEM((P_PER_W, D_MODEL), jnp.float32),
            pltpu.VMEM((P_PER_W, D_MODEL), jnp.float32),
            pltpu.SemaphoreType.DMA,
        ],
    )
    def sc_embed(table_hbm, idx_hbm, pe_hbm, out_hbm, idx_v, rows_v, pe_v, sem):
        wid = lax.axis_index("s") * 2 + lax.axis_index("c")
        pltpu.sync_copy(pe_hbm.at[pl.ds(wid * P_PER_W, P_PER_W)], pe_v)

        @pl.loop(0, BATCH)
        def _(b):
            base = b * MAX_SEQ + wid * P_PER_W
            pltpu.sync_copy(idx_hbm.at[pl.ds(base, P_PER_W)], idx_v)
            pltpu.async_copy(table_hbm.at[idx_v], rows_v, sem).wait()

            @pl.loop(0, P_PER_W)
            def _(r):
                @pl.loop(0, D_MODEL, step=16)
                def _(c):
                    slc = (pl.ds(r, 1), pl.ds(c, 16))
                    rows_v.at[slc][...] = (
                        rows_v.at[slc][...] + pe_v.at[slc][...]
                    )

            pltpu.sync_copy(rows_v, out_hbm.at[pl.ds(base, P_PER_W)])

    out = sc_embed(emb_table, idx, pe)
    return out.reshape(BATCH, MAX_SEQ, D_MODEL)


# speedup vs baseline: 1.6991x; 1.6991x over previous
import functools

import numpy as np

import jax
from jax import lax
import jax.numpy as jnp
from jax.experimental import pallas as pl
from jax.experimental.pallas import tpu as pltpu
from jax.experimental.pallas import tpu_sc as plsc

BATCH = 64
MAX_SEQ = 2048
D_MODEL = 768
VOCAB = 44
NTOK = BATCH * MAX_SEQ
NW = 32
P_PER_W = MAX_SEQ // NW


def _positional_encoding(d_model, max_len):
    position = jnp.arange(0, max_len, dtype=jnp.float32)[:, None]
    div_term = jnp.exp(
        jnp.arange(0, d_model, 2, dtype=jnp.float32) * (-np.log(10000.0) / d_model)
    )
    pe = jnp.zeros((max_len, d_model), dtype=jnp.float32)
    pe = pe.at[:, 0::2].set(jnp.sin(position * div_term))
    pe = pe.at[:, 1::2].set(jnp.cos(position * div_term))
    return pe


@functools.partial(jax.jit, static_argnums=())
def kernel(tokens, emb_table):
    pe = _positional_encoding(D_MODEL, MAX_SEQ)
    idx = tokens.reshape(NTOK)

    mesh = plsc.VectorSubcoreMesh(core_axis_name="c", subcore_axis_name="s")

    @functools.partial(
        pl.kernel,
        mesh=mesh,
        out_type=jax.ShapeDtypeStruct((NTOK, D_MODEL), jnp.float32),
        scratch_types=[
            pltpu.VMEM((P_PER_W,), jnp.int32),
            pltpu.VMEM((P_PER_W, D_MODEL), jnp.float32),
            pltpu.VMEM((P_PER_W, D_MODEL), jnp.float32),
            pltpu.SemaphoreType.DMA,
        ],
    )
    def sc_embed(table_hbm, idx_hbm, pe_hbm, out_hbm, idx_v, rows_v, pe_v, sem):
        wid = lax.axis_index("s") * 2 + lax.axis_index("c")
        pltpu.sync_copy(pe_hbm.at[pl.ds(wid * P_PER_W, P_PER_W)], pe_v)

        @pl.loop(0, BATCH)
        def _(b):
            base = b * MAX_SEQ + wid * P_PER_W
            pltpu.sync_copy(idx_hbm.at[pl.ds(base, P_PER_W)], idx_v)
            pltpu.async_copy(table_hbm.at[idx_v], rows_v, sem).wait()

            @pl.loop(0, P_PER_W)
            def _(r):
                for c in range(0, D_MODEL, 16):
                    slc = (pl.ds(r, 1), pl.ds(c, 16))
                    rows_v.at[slc][...] = (
                        rows_v.at[slc][...] + pe_v.at[slc][...]
                    )

            pltpu.sync_copy(rows_v, out_hbm.at[pl.ds(base, P_PER_W)])

    out = sc_embed(emb_table, idx, pe)
    return out.reshape(BATCH, MAX_SEQ, D_MODEL)
